# baseline (speedup 1.0000x reference)
import functools

import jax
import jax.numpy as jnp
from jax import lax
from jax.experimental import pallas as pl
from jax.experimental.pallas import tpu as pltpu
from jax.experimental.pallas import tpu_sc as plsc

N = 10000
E = 320000
F = 32
NROW = N * F // 128
NC = 2
NS = 16
CHUNK = 128
NCHUNKS = E // CHUNK
NBUF = 4
RPTF = N // NS

CPT0 = 84
_C1TOT = NCHUNKS - NS * CPT0
CPT1HI = _C1TOT // NS + 1
CPT1LO = _C1TOT // NS
NHI = _C1TOT - NS * CPT1LO
_CORE1_BASE = NS * CPT0


def _mesh():
    return plsc.VectorSubcoreMesh(core_axis_name="c", subcore_axis_name="s")


_SC_PARAMS = pltpu.CompilerParams(use_tc_tiling_on_sc=False)


def _per_tile(c, s, run):
    @pl.when(c == 0)
    def _():
        run(CPT0, s * CPT0)

    @pl.when(jnp.logical_and(c != 0, s < NHI))
    def _():
        run(CPT1HI, _CORE1_BASE + s * CPT1HI)

    @pl.when(jnp.logical_and(c != 0, s >= NHI))
    def _():
        run(CPT1LO, _CORE1_BASE + NHI * CPT1HI + (s - NHI) * CPT1LO)


def _sc_degree(e3, ones):
    @functools.partial(
        pl.kernel,
        out_type=jax.ShapeDtypeStruct((NC, N, F), jnp.float32),
        mesh=_mesh(),
        compiler_params=_SC_PARAMS,
        scratch_types=[
            pltpu.VMEM((CPT0, CHUNK), jnp.int32),
            pltpu.VMEM((CHUNK, F), jnp.float32),
            pltpu.VMEM_SHARED((N, F), jnp.float32),
            pltpu.SemaphoreType.DMA,
        ],
    )
    def k(e_hbm, ones_hbm, out_hbm, dstv, onev, dacc, sem):
        c = lax.axis_index("c")
        s = lax.axis_index("s")
        base = s * RPTF
        pltpu.sync_copy(ones_hbm.at[pl.ds(base, RPTF)], dacc.at[pl.ds(base, RPTF)])
        pltpu.sync_copy(ones_hbm.at[pl.ds(0, CHUNK)], onev)

        def run(cpt, cbase):
            pltpu.sync_copy(e_hbm.at[1, pl.ds(cbase, cpt)], dstv.at[pl.ds(0, cpt)])
            plsc.subcore_barrier()
            main = (cpt // NBUF) * NBUF

            @pl.loop(0, main, step=NBUF)
            def _(j):
                descs = []
                for t in range(NBUF):
                    descs.append(
                        pltpu.async_copy(onev, dacc.at[dstv.at[j + t]], sem, add=True)
                    )
                for d in descs:
                    d.wait()

            for jj in range(main, cpt):
                pltpu.sync_copy(onev, dacc.at[dstv.at[jj]], add=True)

        _per_tile(c, s, run)
        plsc.subcore_barrier()
        pltpu.sync_copy(dacc.at[pl.ds(base, RPTF)], out_hbm.at[c, pl.ds(base, RPTF)])

    return k(e3, ones)


def _sc_edge(y, e3):
    @functools.partial(
        pl.kernel,
        out_type=jax.ShapeDtypeStruct((NC, N, F), jnp.float32),
        mesh=_mesh(),
        compiler_params=_SC_PARAMS,
        scratch_types=[
            pltpu.VMEM((CPT0, CHUNK), jnp.int32),
            pltpu.VMEM((CPT0, CHUNK), jnp.int32),
            pltpu.VMEM((NBUF, CHUNK, F), jnp.float32),
            pltpu.VMEM_SHARED((N, F), jnp.float32),
            [pltpu.SemaphoreType.DMA] * NBUF,
            [pltpu.SemaphoreType.DMA] * NBUF,
        ],
    )
    def k(y_hbm, e_hbm, out_hbm, srcv, dstv, rows, acc, gsems, ssems):
        c = lax.axis_index("c")
        s = lax.axis_index("s")
        base = s * RPTF
        pltpu.sync_copy(y_hbm.at[pl.ds(base, RPTF)], acc.at[pl.ds(base, RPTF)])

        def run(cpt, cbase):
            pltpu.sync_copy(e_hbm.at[0, pl.ds(cbase, cpt)], srcv.at[pl.ds(0, cpt)])
            pltpu.sync_copy(e_hbm.at[1, pl.ds(cbase, cpt)], dstv.at[pl.ds(0, cpt)])
            plsc.subcore_barrier()
            for b in range(NBUF):
                pltpu.async_copy(y_hbm.at[srcv.at[b]], rows.at[b], gsems[b])
            main = (cpt // NBUF) * NBUF

            @pl.loop(0, main, step=NBUF)
            def _(j):
                for b in range(NBUF):
                    jj = j + b
                    pltpu.make_async_copy(
                        y_hbm.at[srcv.at[jj]], rows.at[b], gsems[b]
                    ).wait()
                    pltpu.async_copy(
                        rows.at[b], acc.at[dstv.at[jj]], ssems[b], add=True
                    )
                    nxt = jj + 1
                    bn = (b + 1) % NBUF

                    @pl.when(jnp.logical_and(nxt >= NBUF, nxt < cpt))
                    def _():
                        pltpu.make_async_copy(
                            rows.at[bn], acc.at[dstv.at[nxt - NBUF]], ssems[bn]
                        ).wait()
                        pltpu.async_copy(
                            y_hbm.at[srcv.at[nxt]], rows.at[bn], gsems[bn]
                        )

            for jj in range(main, cpt):
                b = jj % NBUF
                pltpu.make_async_copy(y_hbm.at[srcv.at[jj]], rows.at[b], gsems[b]).wait()
                pltpu.async_copy(rows.at[b], acc.at[dstv.at[jj]], ssems[b], add=True)
                nxt = jj + 1
                bn = nxt % NBUF
                if NBUF <= nxt < cpt:
                    pltpu.make_async_copy(
                        rows.at[bn], acc.at[dstv.at[nxt - NBUF]], ssems[bn]
                    ).wait()
                    pltpu.async_copy(y_hbm.at[srcv.at[nxt]], rows.at[bn], gsems[bn])
            for jj in range(max(0, cpt - NBUF), cpt):
                b = jj % NBUF
                pltpu.make_async_copy(rows.at[b], acc.at[dstv.at[jj]], ssems[b]).wait()

        _per_tile(c, s, run)
        plsc.subcore_barrier()
        pltpu.sync_copy(acc.at[pl.ds(base, RPTF)], out_hbm.at[c, pl.ds(base, RPTF)])

    return k(y, e3)


def _blockdiag(w_ref, out_ref, copies):
    kk, nn = w_ref.shape
    out_ref[...] = jnp.zeros(out_ref.shape, jnp.float32)
    for j in range(copies):
        out_ref[pl.ds(j * kk, kk), pl.ds(j * nn, nn)] = w_ref[...]


def _tc1(x, w1, degx):
    def body(x_ref, w1_ref, deg_ref, dis_ref, y_ref, w1b):
        _blockdiag(w1_ref, w1b, 4)
        dis = lax.rsqrt(deg_ref[0] + deg_ref[1] - 1.0)
        dis_ref[...] = dis
        x4 = jnp.reshape(x_ref[...], (NROW, 512))
        xw = jnp.dot(x4, w1b[...], preferred_element_type=jnp.float32)
        y_ref[...] = xw * dis

    return pl.pallas_call(
        body,
        out_shape=(
            jax.ShapeDtypeStruct((NROW, 128), jnp.float32),
            jax.ShapeDtypeStruct((NROW, 128), jnp.float32),
        ),
        scratch_shapes=[pltpu.VMEM((512, 128), jnp.float32)],
    )(x, w1, degx)


def _tc_mid(px, y1x, dis, b1, w2):
    def body(p_ref, y1_ref, dis_ref, b1_ref, w2_ref, y2_ref, w2b):
        _blockdiag(w2_ref, w2b, 4)
        dis = dis_ref[...]
        b128 = jnp.concatenate([b1_ref[...]] * 4)
        h = dis * (p_ref[0] + p_ref[1] - y1_ref[...]) + b128
        h = jnp.maximum(h, 0.0)
        y2_ref[...] = jnp.dot(h, w2b[...], preferred_element_type=jnp.float32) * dis

    return pl.pallas_call(
        body,
        out_shape=jax.ShapeDtypeStruct((NROW, 128), jnp.float32),
        scratch_shapes=[pltpu.VMEM((128, 128), jnp.float32)],
    )(px, y1x, dis, b1, w2)


def _tc_final(qx, y2x, dis, b2, wc, bc):
    def body(q_ref, y2_ref, dis_ref, b2_ref, wc_ref, bc_ref, h2_ref, o_ref, wcb):
        _blockdiag(wc_ref, wcb, 4)
        b128 = jnp.concatenate([b2_ref[...]] * 4)
        h2x = dis_ref[...] * (q_ref[0] + q_ref[1] - y2_ref[...]) + b128
        h2_ref[...] = h2x
        bc8 = jnp.concatenate([bc_ref[...]] * 4)
        o_ref[...] = (
            jnp.dot(h2x, wcb[...], preferred_element_type=jnp.float32) + bc8
        )

    return pl.pallas_call(
        body,
        out_shape=(
            jax.ShapeDtypeStruct((NROW, 128), jnp.float32),
            jax.ShapeDtypeStruct((NROW, 8), jnp.float32),
        ),
        scratch_shapes=[pltpu.VMEM((128, 8), jnp.float32)],
    )(qx, y2x, dis, b2, wc, bc)


def kernel(x, edge_index, W1, b1, W2, b2, Wc, bc):
    e3 = edge_index.astype(jnp.int32).reshape(2, NCHUNKS, CHUNK)
    ones = jnp.ones((NROW, 128), jnp.float32).reshape(N, F)

    deg2 = _sc_degree(e3, ones)
    degx = deg2.reshape(NC, NROW, 128)
    dis, y1x = _tc1(x, W1, degx)
    p = _sc_edge(y1x.reshape(N, F), e3)
    y2x = _tc_mid(p.reshape(NC, NROW, 128), y1x, dis, b1, W2)
    q = _sc_edge(y2x.reshape(N, F), e3)
    h2x, outx = _tc_final(q.reshape(NC, NROW, 128), y2x, dis, b2, Wc, bc)
    return (outx.reshape(N, 2), h2x.reshape(N, F))


# speedup vs baseline: 1.5016x; 1.5016x over previous
import functools

import jax
import jax.numpy as jnp
from jax import lax
from jax.experimental import pallas as pl
from jax.experimental.pallas import tpu as pltpu
from jax.experimental.pallas import tpu_sc as plsc

N = 10000
E = 320000
F = 32
NROW = N * F // 128
NC = 2
NS = 16
CHUNK = 128
NCHUNKS = E // CHUNK
NBUF = 4
RPTF = N // NS

CPT0 = 84
_C1TOT = NCHUNKS - NS * CPT0
CPT1HI = _C1TOT // NS + 1
CPT1LO = _C1TOT // NS
NHI = _C1TOT - NS * CPT1LO
_CORE1_BASE = NS * CPT0


def _mesh():
    return plsc.VectorSubcoreMesh(core_axis_name="c", subcore_axis_name="s")


_SC_PARAMS = pltpu.CompilerParams(use_tc_tiling_on_sc=False)


def _per_tile(c, s, run):
    @pl.when(c == 0)
    def _():
        run(CPT0, s * CPT0)

    @pl.when(jnp.logical_and(c != 0, s < NHI))
    def _():
        run(CPT1HI, _CORE1_BASE + s * CPT1HI)

    @pl.when(jnp.logical_and(c != 0, s >= NHI))
    def _():
        run(CPT1LO, _CORE1_BASE + NHI * CPT1HI + (s - NHI) * CPT1LO)


def _sc_degree(e3, ones):
    @functools.partial(
        pl.kernel,
        out_type=jax.ShapeDtypeStruct((NC, N, F), jnp.float32),
        mesh=_mesh(),
        compiler_params=_SC_PARAMS,
        scratch_types=[
            pltpu.VMEM((CPT0, CHUNK), jnp.int32),
            pltpu.VMEM((CHUNK, F), jnp.float32),
            pltpu.VMEM_SHARED((N, F), jnp.float32),
            pltpu.SemaphoreType.DMA,
        ],
    )
    def k(e_hbm, ones_hbm, out_hbm, dstv, onev, dacc, sem):
        c = lax.axis_index("c")
        s = lax.axis_index("s")
        base = s * RPTF
        pltpu.sync_copy(ones_hbm.at[pl.ds(base, RPTF)], dacc.at[pl.ds(base, RPTF)])
        pltpu.sync_copy(ones_hbm.at[pl.ds(0, CHUNK)], onev)

        def run(cpt, cbase):
            pltpu.sync_copy(e_hbm.at[1, pl.ds(cbase, cpt)], dstv.at[pl.ds(0, cpt)])
            plsc.subcore_barrier()
            main = (cpt // NBUF) * NBUF

            @pl.loop(0, main, step=NBUF)
            def _(j):
                descs = []
                for t in range(NBUF):
                    descs.append(
                        pltpu.async_copy(onev, dacc.at[dstv.at[j + t]], sem, add=True)
                    )
                for d in descs:
                    d.wait()

            for jj in range(main, cpt):
                pltpu.sync_copy(onev, dacc.at[dstv.at[jj]], add=True)

        _per_tile(c, s, run)
        plsc.subcore_barrier()
        pltpu.sync_copy(dacc.at[pl.ds(base, RPTF)], out_hbm.at[c, pl.ds(base, RPTF)])

    return k(e3, ones)


def _sc_edge(y, e3):
    @functools.partial(
        pl.kernel,
        out_type=jax.ShapeDtypeStruct((NC, N, F), jnp.float32),
        mesh=_mesh(),
        compiler_params=_SC_PARAMS,
        scratch_types=[
            pltpu.VMEM((CPT0, CHUNK), jnp.int32),
            pltpu.VMEM((CPT0, CHUNK), jnp.int32),
            pltpu.VMEM((NBUF, CHUNK, F), jnp.float32),
            pltpu.VMEM_SHARED((N, F), jnp.float32),
            [pltpu.SemaphoreType.DMA] * NBUF,
            [pltpu.SemaphoreType.DMA] * NBUF,
        ],
    )
    def k(y_hbm, e_hbm, out_hbm, srcv, dstv, rows, acc, gsems, ssems):
        c = lax.axis_index("c")
        s = lax.axis_index("s")
        base = s * RPTF
        pltpu.sync_copy(y_hbm.at[pl.ds(base, RPTF)], acc.at[pl.ds(base, RPTF)])

        def run(cpt, cbase):
            pltpu.sync_copy(e_hbm.at[0, pl.ds(cbase, cpt)], srcv.at[pl.ds(0, cpt)])
            pltpu.sync_copy(e_hbm.at[1, pl.ds(cbase, cpt)], dstv.at[pl.ds(0, cpt)])
            plsc.subcore_barrier()
            for b in range(NBUF):
                pltpu.async_copy(y_hbm.at[srcv.at[b]], rows.at[b], gsems[b])
            main = (cpt // NBUF) * NBUF

            @pl.loop(0, main, step=NBUF)
            def _(j):
                for b in range(NBUF):
                    jj = j + b
                    pltpu.make_async_copy(
                        y_hbm.at[srcv.at[jj]], rows.at[b], gsems[b]
                    ).wait()
                    pltpu.async_copy(
                        rows.at[b], acc.at[dstv.at[jj]], ssems[b], add=True
                    )

                    @pl.when(jj + NBUF < cpt)
                    def _():
                        pltpu.make_async_copy(
                            rows.at[b], acc.at[dstv.at[jj]], ssems[b]
                        ).wait()
                        pltpu.async_copy(
                            y_hbm.at[srcv.at[jj + NBUF]], rows.at[b], gsems[b]
                        )

            for jj in range(main, cpt):
                b = jj % NBUF
                pltpu.make_async_copy(y_hbm.at[srcv.at[jj]], rows.at[b], gsems[b]).wait()
                pltpu.async_copy(rows.at[b], acc.at[dstv.at[jj]], ssems[b], add=True)
            for jj in range(max(0, cpt - NBUF), cpt):
                b = jj % NBUF
                pltpu.make_async_copy(rows.at[b], acc.at[dstv.at[jj]], ssems[b]).wait()

        _per_tile(c, s, run)
        plsc.subcore_barrier()
        pltpu.sync_copy(acc.at[pl.ds(base, RPTF)], out_hbm.at[c, pl.ds(base, RPTF)])

    return k(y, e3)


def _blockdiag(w_ref, out_ref, copies):
    kk, nn = w_ref.shape
    out_ref[...] = jnp.zeros(out_ref.shape, jnp.float32)
    for j in range(copies):
        out_ref[pl.ds(j * kk, kk), pl.ds(j * nn, nn)] = w_ref[...]


def _tc1(x, w1, degx):
    def body(x_ref, w1_ref, deg_ref, dis_ref, y_ref, w1b):
        _blockdiag(w1_ref, w1b, 4)
        dis = lax.rsqrt(deg_ref[0] + deg_ref[1] - 1.0)
        dis_ref[...] = dis
        x4 = jnp.reshape(x_ref[...], (NROW, 512))
        xw = jnp.dot(x4, w1b[...], preferred_element_type=jnp.float32)
        y_ref[...] = xw * dis

    return pl.pallas_call(
        body,
        out_shape=(
            jax.ShapeDtypeStruct((NROW, 128), jnp.float32),
            jax.ShapeDtypeStruct((NROW, 128), jnp.float32),
        ),
        scratch_shapes=[pltpu.VMEM((512, 128), jnp.float32)],
    )(x, w1, degx)


def _tc_mid(px, y1x, dis, b1, w2):
    def body(p_ref, y1_ref, dis_ref, b1_ref, w2_ref, y2_ref, w2b):
        _blockdiag(w2_ref, w2b, 4)
        dis = dis_ref[...]
        b128 = jnp.concatenate([b1_ref[...]] * 4)
        h = dis * (p_ref[0] + p_ref[1] - y1_ref[...]) + b128
        h = jnp.maximum(h, 0.0)
        y2_ref[...] = jnp.dot(h, w2b[...], preferred_element_type=jnp.float32) * dis

    return pl.pallas_call(
        body,
        out_shape=jax.ShapeDtypeStruct((NROW, 128), jnp.float32),
        scratch_shapes=[pltpu.VMEM((128, 128), jnp.float32)],
    )(px, y1x, dis, b1, w2)


def _tc_final(qx, y2x, dis, b2, wc, bc):
    def body(q_ref, y2_ref, dis_ref, b2_ref, wc_ref, bc_ref, h2_ref, o_ref, wcb):
        _blockdiag(wc_ref, wcb, 4)
        b128 = jnp.concatenate([b2_ref[...]] * 4)
        h2x = dis_ref[...] * (q_ref[0] + q_ref[1] - y2_ref[...]) + b128
        h2_ref[...] = h2x
        bc8 = jnp.concatenate([bc_ref[...]] * 4)
        o_ref[...] = (
            jnp.dot(h2x, wcb[...], preferred_element_type=jnp.float32) + bc8
        )

    return pl.pallas_call(
        body,
        out_shape=(
            jax.ShapeDtypeStruct((NROW, 128), jnp.float32),
            jax.ShapeDtypeStruct((NROW, 8), jnp.float32),
        ),
        scratch_shapes=[pltpu.VMEM((128, 8), jnp.float32)],
    )(qx, y2x, dis, b2, wc, bc)


def kernel(x, edge_index, W1, b1, W2, b2, Wc, bc):
    e3 = edge_index.astype(jnp.int32).reshape(2, NCHUNKS, CHUNK)
    ones = jnp.ones((NROW, 128), jnp.float32).reshape(N, F)

    deg2 = _sc_degree(e3, ones)
    degx = deg2.reshape(NC, NROW, 128)
    dis, y1x = _tc1(x, W1, degx)
    p = _sc_edge(y1x.reshape(N, F), e3)
    y2x = _tc_mid(p.reshape(NC, NROW, 128), y1x, dis, b1, W2)
    q = _sc_edge(y2x.reshape(N, F), e3)
    h2x, outx = _tc_final(q.reshape(NC, NROW, 128), y2x, dis, b2, Wc, bc)
    return (outx.reshape(N, 2), h2x.reshape(N, F))


# speedup vs baseline: 1.5480x; 1.0309x over previous
import functools

import jax
import jax.numpy as jnp
from jax import lax
from jax.experimental import pallas as pl
from jax.experimental.pallas import tpu as pltpu
from jax.experimental.pallas import tpu_sc as plsc

N = 10000
E = 320000
F = 32
NROW = N * F // 128
NC = 2
NS = 16
CHUNK = 128
NCHUNKS = E // CHUNK
NBUF = 4
RPTF = N // NS

CPT0 = 81
_C1TOT = NCHUNKS - NS * CPT0
CPT1HI = _C1TOT // NS + 1
CPT1LO = _C1TOT // NS
NHI = _C1TOT - NS * CPT1LO
_CORE1_BASE = NS * CPT0


def _mesh():
    return plsc.VectorSubcoreMesh(core_axis_name="c", subcore_axis_name="s")


_SC_PARAMS = pltpu.CompilerParams(use_tc_tiling_on_sc=False)


def _per_tile(c, s, run):
    @pl.when(c == 0)
    def _():
        run(CPT0, s * CPT0)

    @pl.when(jnp.logical_and(c != 0, s < NHI))
    def _():
        run(CPT1HI, _CORE1_BASE + s * CPT1HI)

    @pl.when(jnp.logical_and(c != 0, s >= NHI))
    def _():
        run(CPT1LO, _CORE1_BASE + NHI * CPT1HI + (s - NHI) * CPT1LO)


def _sc_degree(e3, ones):
    @functools.partial(
        pl.kernel,
        out_type=jax.ShapeDtypeStruct((NC, N, F), jnp.float32),
        mesh=_mesh(),
        compiler_params=_SC_PARAMS,
        scratch_types=[
            pltpu.VMEM((CPT0, CHUNK), jnp.int32),
            pltpu.VMEM((CHUNK, F), jnp.float32),
            pltpu.VMEM_SHARED((N, F), jnp.float32),
            pltpu.SemaphoreType.DMA,
        ],
    )
    def k(e_hbm, ones_hbm, out_hbm, dstv, onev, dacc, sem):
        c = lax.axis_index("c")
        s = lax.axis_index("s")
        base = s * RPTF
        pltpu.sync_copy(ones_hbm.at[pl.ds(base, RPTF)], dacc.at[pl.ds(base, RPTF)])
        pltpu.sync_copy(ones_hbm.at[pl.ds(0, CHUNK)], onev)

        def run(cpt, cbase):
            pltpu.sync_copy(e_hbm.at[1, pl.ds(cbase, cpt)], dstv.at[pl.ds(0, cpt)])
            plsc.subcore_barrier()
            main = (cpt // NBUF) * NBUF

            @pl.loop(0, main, step=NBUF)
            def _(j):
                descs = []
                for t in range(NBUF):
                    descs.append(
                        pltpu.async_copy(onev, dacc.at[dstv.at[j + t]], sem, add=True)
                    )
                for d in descs:
                    d.wait()

            for jj in range(main, cpt):
                pltpu.sync_copy(onev, dacc.at[dstv.at[jj]], add=True)

        _per_tile(c, s, run)
        plsc.subcore_barrier()
        pltpu.sync_copy(dacc.at[pl.ds(base, RPTF)], out_hbm.at[c, pl.ds(base, RPTF)])

    return k(e3, ones)


def _sc_edge(y, e3):
    @functools.partial(
        pl.kernel,
        out_type=jax.ShapeDtypeStruct((NC, N, F), jnp.float32),
        mesh=_mesh(),
        compiler_params=_SC_PARAMS,
        scratch_types=[
            pltpu.VMEM((CPT0, CHUNK), jnp.int32),
            pltpu.VMEM((CPT0, CHUNK), jnp.int32),
            pltpu.VMEM((NBUF, CHUNK, F), jnp.float32),
            pltpu.VMEM_SHARED((N, F), jnp.float32),
            [pltpu.SemaphoreType.DMA] * NBUF,
            [pltpu.SemaphoreType.DMA] * NBUF,
        ],
    )
    def k(y_hbm, e_hbm, out_hbm, srcv, dstv, rows, acc, gsems, ssems):
        c = lax.axis_index("c")
        s = lax.axis_index("s")
        base = s * RPTF
        pltpu.sync_copy(y_hbm.at[pl.ds(base, RPTF)], acc.at[pl.ds(base, RPTF)])

        def run(cpt, cbase):
            pltpu.sync_copy(e_hbm.at[0, pl.ds(cbase, cpt)], srcv.at[pl.ds(0, cpt)])
            pltpu.sync_copy(e_hbm.at[1, pl.ds(cbase, cpt)], dstv.at[pl.ds(0, cpt)])
            plsc.subcore_barrier()
            for b in range(NBUF):
                pltpu.async_copy(y_hbm.at[srcv.at[b]], rows.at[b], gsems[b])
            main = (cpt // NBUF) * NBUF

            @pl.loop(0, main, step=NBUF)
            def _(j):
                for b in range(NBUF):
                    jj = j + b
                    pltpu.make_async_copy(
                        y_hbm.at[srcv.at[jj]], rows.at[b], gsems[b]
                    ).wait()
                    pltpu.async_copy(
                        rows.at[b], acc.at[dstv.at[jj]], ssems[b], add=True
                    )

                    @pl.when(jj + NBUF < cpt)
                    def _():
                        pltpu.make_async_copy(
                            rows.at[b], acc.at[dstv.at[jj]], ssems[b]
                        ).wait()
                        pltpu.async_copy(
                            y_hbm.at[srcv.at[jj + NBUF]], rows.at[b], gsems[b]
                        )

            for jj in range(main, cpt):
                b = jj % NBUF
                pltpu.make_async_copy(y_hbm.at[srcv.at[jj]], rows.at[b], gsems[b]).wait()
                pltpu.async_copy(rows.at[b], acc.at[dstv.at[jj]], ssems[b], add=True)
            for jj in range(max(0, cpt - NBUF), cpt):
                b = jj % NBUF
                pltpu.make_async_copy(rows.at[b], acc.at[dstv.at[jj]], ssems[b]).wait()

        _per_tile(c, s, run)
        plsc.subcore_barrier()
        pltpu.sync_copy(acc.at[pl.ds(base, RPTF)], out_hbm.at[c, pl.ds(base, RPTF)])

    return k(y, e3)


def _blockdiag(w_ref, out_ref, copies):
    kk, nn = w_ref.shape
    out_ref[...] = jnp.zeros(out_ref.shape, jnp.float32)
    for j in range(copies):
        out_ref[pl.ds(j * kk, kk), pl.ds(j * nn, nn)] = w_ref[...]


def _tc1(x, w1, degx):
    def body(x_ref, w1_ref, deg_ref, dis_ref, y_ref, w1b):
        _blockdiag(w1_ref, w1b, 4)
        dis = lax.rsqrt(deg_ref[0] + deg_ref[1] - 1.0)
        dis_ref[...] = dis
        x4 = jnp.reshape(x_ref[...], (NROW, 512))
        xw = jnp.dot(x4, w1b[...], preferred_element_type=jnp.float32)
        y_ref[...] = xw * dis

    return pl.pallas_call(
        body,
        out_shape=(
            jax.ShapeDtypeStruct((NROW, 128), jnp.float32),
            jax.ShapeDtypeStruct((NROW, 128), jnp.float32),
        ),
        scratch_shapes=[pltpu.VMEM((512, 128), jnp.float32)],
    )(x, w1, degx)


def _tc_mid(px, y1x, dis, b1, w2):
    def body(p_ref, y1_ref, dis_ref, b1_ref, w2_ref, y2_ref, w2b):
        _blockdiag(w2_ref, w2b, 4)
        dis = dis_ref[...]
        b128 = jnp.concatenate([b1_ref[...]] * 4)
        h = dis * (p_ref[0] + p_ref[1] - y1_ref[...]) + b128
        h = jnp.maximum(h, 0.0)
        y2_ref[...] = jnp.dot(h, w2b[...], preferred_element_type=jnp.float32) * dis

    return pl.pallas_call(
        body,
        out_shape=jax.ShapeDtypeStruct((NROW, 128), jnp.float32),
        scratch_shapes=[pltpu.VMEM((128, 128), jnp.float32)],
    )(px, y1x, dis, b1, w2)


def _tc_final(qx, y2x, dis, b2, wc, bc):
    def body(q_ref, y2_ref, dis_ref, b2_ref, wc_ref, bc_ref, h2_ref, o_ref, wcb):
        _blockdiag(wc_ref, wcb, 4)
        b128 = jnp.concatenate([b2_ref[...]] * 4)
        h2x = dis_ref[...] * (q_ref[0] + q_ref[1] - y2_ref[...]) + b128
        h2_ref[...] = h2x
        bc8 = jnp.concatenate([bc_ref[...]] * 4)
        o_ref[...] = (
            jnp.dot(h2x, wcb[...], preferred_element_type=jnp.float32) + bc8
        )

    return pl.pallas_call(
        body,
        out_shape=(
            jax.ShapeDtypeStruct((NROW, 128), jnp.float32),
            jax.ShapeDtypeStruct((NROW, 8), jnp.float32),
        ),
        scratch_shapes=[pltpu.VMEM((128, 8), jnp.float32)],
    )(qx, y2x, dis, b2, wc, bc)


def kernel(x, edge_index, W1, b1, W2, b2, Wc, bc):
    e3 = edge_index.astype(jnp.int32).reshape(2, NCHUNKS, CHUNK)
    ones = jnp.ones((NROW, 128), jnp.float32).reshape(N, F)

    deg2 = _sc_degree(e3, ones)
    degx = deg2.reshape(NC, NROW, 128)
    dis, y1x = _tc1(x, W1, degx)
    p = _sc_edge(y1x.reshape(N, F), e3)
    y2x = _tc_mid(p.reshape(NC, NROW, 128), y1x, dis, b1, W2)
    q = _sc_edge(y2x.reshape(N, F), e3)
    h2x, outx = _tc_final(q.reshape(NC, NROW, 128), y2x, dis, b2, Wc, bc)
    return (outx.reshape(N, 2), h2x.reshape(N, F))


# speedup vs baseline: 1.6154x; 1.0435x over previous
import functools

import jax
import jax.numpy as jnp
from jax import lax
from jax.experimental import pallas as pl
from jax.experimental.pallas import tpu as pltpu
from jax.experimental.pallas import tpu_sc as plsc

N = 10000
E = 320000
F = 32
NROW = N * F // 128
NC = 2
NS = 16
CHUNK = 128
NCHUNKS = E // CHUNK
NBUF = 8
RPTF = N // NS

CPT0 = 81
_C1TOT = NCHUNKS - NS * CPT0
CPT1HI = _C1TOT // NS + 1
CPT1LO = _C1TOT // NS
NHI = _C1TOT - NS * CPT1LO
_CORE1_BASE = NS * CPT0


def _mesh():
    return plsc.VectorSubcoreMesh(core_axis_name="c", subcore_axis_name="s")


_SC_PARAMS = pltpu.CompilerParams(use_tc_tiling_on_sc=False)


def _per_tile(c, s, run):
    @pl.when(c == 0)
    def _():
        run(CPT0, s * CPT0)

    @pl.when(jnp.logical_and(c != 0, s < NHI))
    def _():
        run(CPT1HI, _CORE1_BASE + s * CPT1HI)

    @pl.when(jnp.logical_and(c != 0, s >= NHI))
    def _():
        run(CPT1LO, _CORE1_BASE + NHI * CPT1HI + (s - NHI) * CPT1LO)


def _sc_degree(e3, ones):
    @functools.partial(
        pl.kernel,
        out_type=jax.ShapeDtypeStruct((NC, N, F), jnp.float32),
        mesh=_mesh(),
        compiler_params=_SC_PARAMS,
        scratch_types=[
            pltpu.VMEM((CPT0, CHUNK), jnp.int32),
            pltpu.VMEM((CHUNK, F), jnp.float32),
            pltpu.VMEM_SHARED((N, F), jnp.float32),
            pltpu.SemaphoreType.DMA,
        ],
    )
    def k(e_hbm, ones_hbm, out_hbm, dstv, onev, dacc, sem):
        c = lax.axis_index("c")
        s = lax.axis_index("s")
        base = s * RPTF
        pltpu.sync_copy(ones_hbm.at[pl.ds(base, RPTF)], dacc.at[pl.ds(base, RPTF)])
        pltpu.sync_copy(ones_hbm.at[pl.ds(0, CHUNK)], onev)

        def run(cpt, cbase):
            pltpu.sync_copy(e_hbm.at[1, pl.ds(cbase, cpt)], dstv.at[pl.ds(0, cpt)])
            plsc.subcore_barrier()
            main = (cpt // NBUF) * NBUF

            @pl.loop(0, main, step=NBUF)
            def _(j):
                descs = []
                for t in range(NBUF):
                    descs.append(
                        pltpu.async_copy(onev, dacc.at[dstv.at[j + t]], sem, add=True)
                    )
                for d in descs:
                    d.wait()

            for jj in range(main, cpt):
                pltpu.sync_copy(onev, dacc.at[dstv.at[jj]], add=True)

        _per_tile(c, s, run)
        plsc.subcore_barrier()
        pltpu.sync_copy(dacc.at[pl.ds(base, RPTF)], out_hbm.at[c, pl.ds(base, RPTF)])

    return k(e3, ones)


def _sc_edge(y, e3):
    @functools.partial(
        pl.kernel,
        out_type=jax.ShapeDtypeStruct((NC, N, F), jnp.float32),
        mesh=_mesh(),
        compiler_params=_SC_PARAMS,
        scratch_types=[
            pltpu.VMEM((CPT0, CHUNK), jnp.int32),
            pltpu.VMEM((CPT0, CHUNK), jnp.int32),
            pltpu.VMEM((NBUF, CHUNK, F), jnp.float32),
            pltpu.VMEM_SHARED((N, F), jnp.float32),
            [pltpu.SemaphoreType.DMA] * NBUF,
            [pltpu.SemaphoreType.DMA] * NBUF,
        ],
    )
    def k(y_hbm, e_hbm, out_hbm, srcv, dstv, rows, acc, gsems, ssems):
        c = lax.axis_index("c")
        s = lax.axis_index("s")
        base = s * RPTF
        pltpu.sync_copy(y_hbm.at[pl.ds(base, RPTF)], acc.at[pl.ds(base, RPTF)])

        def run(cpt, cbase):
            pltpu.sync_copy(e_hbm.at[0, pl.ds(cbase, cpt)], srcv.at[pl.ds(0, cpt)])
            pltpu.sync_copy(e_hbm.at[1, pl.ds(cbase, cpt)], dstv.at[pl.ds(0, cpt)])
            plsc.subcore_barrier()
            for b in range(NBUF):
                pltpu.async_copy(y_hbm.at[srcv.at[b]], rows.at[b], gsems[b])
            main = (cpt // NBUF) * NBUF

            @pl.loop(0, main, step=NBUF)
            def _(j):
                for b in range(NBUF):
                    jj = j + b
                    pltpu.make_async_copy(
                        y_hbm.at[srcv.at[jj]], rows.at[b], gsems[b]
                    ).wait()
                    pltpu.async_copy(
                        rows.at[b], acc.at[dstv.at[jj]], ssems[b], add=True
                    )

                    @pl.when(jj + NBUF < cpt)
                    def _():
                        pltpu.make_async_copy(
                            rows.at[b], acc.at[dstv.at[jj]], ssems[b]
                        ).wait()
                        pltpu.async_copy(
                            y_hbm.at[srcv.at[jj + NBUF]], rows.at[b], gsems[b]
                        )

            for jj in range(main, cpt):
                b = jj % NBUF
                pltpu.make_async_copy(y_hbm.at[srcv.at[jj]], rows.at[b], gsems[b]).wait()
                pltpu.async_copy(rows.at[b], acc.at[dstv.at[jj]], ssems[b], add=True)
            for jj in range(max(0, cpt - NBUF), cpt):
                b = jj % NBUF
                pltpu.make_async_copy(rows.at[b], acc.at[dstv.at[jj]], ssems[b]).wait()

        _per_tile(c, s, run)
        plsc.subcore_barrier()
        pltpu.sync_copy(acc.at[pl.ds(base, RPTF)], out_hbm.at[c, pl.ds(base, RPTF)])

    return k(y, e3)


def _blockdiag(w_ref, out_ref, copies):
    kk, nn = w_ref.shape
    out_ref[...] = jnp.zeros(out_ref.shape, jnp.float32)
    for j in range(copies):
        out_ref[pl.ds(j * kk, kk), pl.ds(j * nn, nn)] = w_ref[...]


def _tc1(x, w1, degx):
    def body(x_ref, w1_ref, deg_ref, dis_ref, y_ref, w1b):
        _blockdiag(w1_ref, w1b, 4)
        dis = lax.rsqrt(deg_ref[0] + deg_ref[1] - 1.0)
        dis_ref[...] = dis
        x4 = jnp.reshape(x_ref[...], (NROW, 512))
        xw = jnp.dot(x4, w1b[...], preferred_element_type=jnp.float32)
        y_ref[...] = xw * dis

    return pl.pallas_call(
        body,
        out_shape=(
            jax.ShapeDtypeStruct((NROW, 128), jnp.float32),
            jax.ShapeDtypeStruct((NROW, 128), jnp.float32),
        ),
        scratch_shapes=[pltpu.VMEM((512, 128), jnp.float32)],
    )(x, w1, degx)


def _tc_mid(px, y1x, dis, b1, w2):
    def body(p_ref, y1_ref, dis_ref, b1_ref, w2_ref, y2_ref, w2b):
        _blockdiag(w2_ref, w2b, 4)
        dis = dis_ref[...]
        b128 = jnp.concatenate([b1_ref[...]] * 4)
        h = dis * (p_ref[0] + p_ref[1] - y1_ref[...]) + b128
        h = jnp.maximum(h, 0.0)
        y2_ref[...] = jnp.dot(h, w2b[...], preferred_element_type=jnp.float32) * dis

    return pl.pallas_call(
        body,
        out_shape=jax.ShapeDtypeStruct((NROW, 128), jnp.float32),
        scratch_shapes=[pltpu.VMEM((128, 128), jnp.float32)],
    )(px, y1x, dis, b1, w2)


def _tc_final(qx, y2x, dis, b2, wc, bc):
    def body(q_ref, y2_ref, dis_ref, b2_ref, wc_ref, bc_ref, h2_ref, o_ref, wcb):
        _blockdiag(wc_ref, wcb, 4)
        b128 = jnp.concatenate([b2_ref[...]] * 4)
        h2x = dis_ref[...] * (q_ref[0] + q_ref[1] - y2_ref[...]) + b128
        h2_ref[...] = h2x
        bc8 = jnp.concatenate([bc_ref[...]] * 4)
        o_ref[...] = (
            jnp.dot(h2x, wcb[...], preferred_element_type=jnp.float32) + bc8
        )

    return pl.pallas_call(
        body,
        out_shape=(
            jax.ShapeDtypeStruct((NROW, 128), jnp.float32),
            jax.ShapeDtypeStruct((NROW, 8), jnp.float32),
        ),
        scratch_shapes=[pltpu.VMEM((128, 8), jnp.float32)],
    )(qx, y2x, dis, b2, wc, bc)


def kernel(x, edge_index, W1, b1, W2, b2, Wc, bc):
    e3 = edge_index.astype(jnp.int32).reshape(2, NCHUNKS, CHUNK)
    ones = jnp.ones((NROW, 128), jnp.float32).reshape(N, F)

    deg2 = _sc_degree(e3, ones)
    degx = deg2.reshape(NC, NROW, 128)
    dis, y1x = _tc1(x, W1, degx)
    p = _sc_edge(y1x.reshape(N, F), e3)
    y2x = _tc_mid(p.reshape(NC, NROW, 128), y1x, dis, b1, W2)
    q = _sc_edge(y2x.reshape(N, F), e3)
    h2x, outx = _tc_final(q.reshape(NC, NROW, 128), y2x, dis, b2, Wc, bc)
    return (outx.reshape(N, 2), h2x.reshape(N, F))
